# baseline (speedup 1.0000x reference)
import functools

import jax
import jax.numpy as jnp
from jax import lax
from jax.experimental import pallas as pl
from jax.experimental.pallas import tpu as pltpu
from jax.experimental.pallas import tpu_sc as plsc

E = 8
D = 1024
F = 2048
T = 4096
A = 2 * T
TILE_M = 256
C = A + E * TILE_M
NT = C // TILE_M
NT_PAD = 48
NC, NS = 2, 16
NW = NC * NS
APW = A // NW
TPW = T // NW
RB = 2048
NRB = T // RB
NCHB = RB // 256
XCH = 32
NXCH = APW // XCH
CH = 16
NCCH = TPW // CH

_sc_mesh = plsc.VectorSubcoreMesh(
    core_axis_name="c", subcore_axis_name="s", num_cores=NC, num_subcores=NS)
_sc_params = pltpu.CompilerParams(needs_layout_passes=False)


def _worker_id():
    return lax.axis_index("s") * NC + lax.axis_index("c")


def _router_body(x_ref, wr_ref, e1_ref, e2_ref, w1_ref, w2_ref, h1_ref, h2_ref):
    x = x_ref[...]
    wr = wr_ref[...]
    logits = jnp.dot(x, wr)
    eidx = lax.broadcasted_iota(jnp.int32, (RB, E), 1)
    m1 = jnp.max(logits, axis=1)
    i1 = jnp.min(jnp.where(logits == m1[:, None], eidx, E), axis=1)
    masked = jnp.where(eidx == i1[:, None], -jnp.inf, logits)
    m2 = jnp.max(masked, axis=1)
    i2 = jnp.min(jnp.where(masked == m2[:, None], eidx, E), axis=1)
    t = jnp.exp(m2 - m1)
    g1 = 1.0 / (1.0 + t)
    e1_ref[...] = i1[:, None]
    e2_ref[...] = i2[:, None]
    w1_ref[...] = g1[:, None]
    w2_ref[...] = (t * g1)[:, None]
    lidx = lax.broadcasted_iota(jnp.int32, (RB, 16), 1)
    oh1 = (lidx == i1[:, None]).astype(jnp.int32)
    oh2 = (lidx == i2[:, None]).astype(jnp.int32)
    h1_ref[...] = jnp.concatenate(
        [jnp.sum(oh1[i * 256:(i + 1) * 256], axis=0).reshape(1, 1, 16)
         for i in range(NCHB)], axis=0)
    h2_ref[...] = jnp.concatenate(
        [jnp.sum(oh2[i * 256:(i + 1) * 256], axis=0).reshape(1, 1, 16)
         for i in range(NCHB)], axis=0)


def _router(x, wr):
    return pl.pallas_call(
        _router_body,
        grid=(NRB,),
        in_specs=[
            pl.BlockSpec((RB, D), lambda m: (m, 0)),
            pl.BlockSpec((D, E), lambda m: (0, 0)),
        ],
        out_specs=[
            pl.BlockSpec((RB, 1), lambda m: (m, 0)),
            pl.BlockSpec((RB, 1), lambda m: (m, 0)),
            pl.BlockSpec((RB, 1), lambda m: (m, 0)),
            pl.BlockSpec((RB, 1), lambda m: (m, 0)),
            pl.BlockSpec((NCHB, 1, 16), lambda m: (m, 0, 0)),
            pl.BlockSpec((NCHB, 1, 16), lambda m: (m, 0, 0)),
        ],
        out_shape=[
            jax.ShapeDtypeStruct((T, 1), jnp.int32),
            jax.ShapeDtypeStruct((T, 1), jnp.int32),
            jax.ShapeDtypeStruct((T, 1), jnp.float32),
            jax.ShapeDtypeStruct((T, 1), jnp.float32),
            jax.ShapeDtypeStruct((NCHB * NRB, 1, 16), jnp.int32),
            jax.ShapeDtypeStruct((NCHB * NRB, 1, 16), jnp.int32),
        ],
    )(x, wr)


def _dispatch_body(hist_hbm, ea_hbm, wa_hbm, x_hbm,
                   xs_hbm, gs_hbm, inv_hbm, te_hbm,
                   allhist_v, eid_v, gate_v, cnt_v, te_v,
                   pos8_v, pos_lin_v, xbuf0_v, xbuf1_v, xbuf2_v,
                   rsem0, rsem1, rsem2, ssem0, ssem1, ssem2):
    wid = _worker_id()
    lane = lax.iota(jnp.int32, 16)
    zeros = jnp.zeros((16,), jnp.int32)
    abase = wid * APW
    tok_base = jnp.where(wid < NS, abase, abase - T)

    rdesc = [None, None, None]
    sdesc = [None, None, None]
    bufs = [xbuf0_v, xbuf1_v, xbuf2_v]
    rsems = [rsem0, rsem1, rsem2]
    ssems = [ssem0, ssem1, ssem2]

    def _start_read(c):
        b = c % 3
        if sdesc[b] is not None:
            sdesc[b].wait()
        rdesc[b] = pltpu.async_copy(
            x_hbm.at[pl.ds(tok_base + c * XCH, XCH)], bufs[b], rsems[b])

    _start_read(0)
    _start_read(1)
    _start_read(2)

    pltpu.sync_copy(hist_hbm, allhist_v)
    totals = zeros
    prefix = zeros
    for w in range(NW):
        hv = allhist_v[w]
        totals = totals + hv
        prefix = prefix + jnp.where(w < wid, hv, zeros)
    padded = ((totals + (TILE_M - 1)) // TILE_M) * TILE_M
    seg_end = plsc.cumsum(padded)
    my_base = (seg_end - padded) + prefix

    pltpu.sync_copy(ea_hbm.at[pl.ds(abase, APW)], eid_v)
    pltpu.sync_copy(wa_hbm.at[pl.ds(abase, APW)], gate_v)

    cntvec = my_base
    for g in range(APW // 16):
        cnt_v[...] = cntvec
        v = eid_v[pl.ds(g * 16, 16)]
        base_e = plsc.load_gather(cnt_v, [v])
        ranks = zeros
        for e in range(E):
            m = v == e
            cs = plsc.cumsum(jnp.where(m, 1, 0))
            ranks = jnp.where(m, cs, ranks)
            tote = plsc.all_reduce_population_count(m)
            cntvec = jnp.where(lane == e, cntvec + tote, cntvec)
        pos = base_e + ranks - 1
        pos8_v[g // 2, pl.ds((g % 2) * 16, 16)] = pos
        pos_lin_v[pl.ds(g * 16, 16)] = pos

    pltpu.sync_copy(pos_lin_v, inv_hbm.at[pl.ds(abase, APW)])

    for c in range(NXCH):
        b = c % 3
        rdesc[b].wait()
        sdesc[b] = pltpu.async_copy(bufs[b], xs_hbm.at[pos8_v.at[c]], ssems[b])
        pltpu.sync_copy(
            gate_v.at[pl.ds(c * XCH, XCH)], gs_hbm.at[pos8_v.at[c]])
        if c + 3 < NXCH:
            _start_read(c + 3)
    for b in range(3):
        if sdesc[b] is not None:
            sdesc[b].wait()

    @pl.when(wid == 0)
    def _():
        cnt_v[...] = (seg_end - padded) + totals
        for g in range(NT_PAD // 16):
            jv = (lane + g * 16) * TILE_M
            te = jnp.zeros((16,), jnp.int32)
            for e in range(E):
                se = seg_end[e]
                te = te + jnp.where(jv >= se, 1, 0)
            te = jnp.minimum(te, E - 1)
            dend = plsc.load_gather(cnt_v, [te])
            te_v[pl.ds(g * 16, 16)] = te + jnp.where(jv >= dend, E, 0)
        pltpu.sync_copy(te_v, te_hbm)


_dispatch = functools.partial(
    pl.kernel,
    out_type=[
        jax.ShapeDtypeStruct((C, D), jnp.float32),
        jax.ShapeDtypeStruct((C,), jnp.float32),
        jax.ShapeDtypeStruct((A,), jnp.int32),
        jax.ShapeDtypeStruct((NT_PAD,), jnp.int32),
    ],
    mesh=_sc_mesh,
    compiler_params=_sc_params,
    scratch_types=[
        pltpu.VMEM((NW, 16), jnp.int32),
        pltpu.VMEM((APW,), jnp.int32),
        pltpu.VMEM((APW,), jnp.float32),
        pltpu.VMEM((16,), jnp.int32),
        pltpu.VMEM((NT_PAD,), jnp.int32),
        pltpu.VMEM((NXCH, XCH), jnp.int32),
        pltpu.VMEM((APW,), jnp.int32),
        pltpu.VMEM((XCH, D), jnp.float32),
        pltpu.VMEM((XCH, D), jnp.float32),
        pltpu.VMEM((XCH, D), jnp.float32),
        pltpu.SemaphoreType.DMA,
        pltpu.SemaphoreType.DMA,
        pltpu.SemaphoreType.DMA,
        pltpu.SemaphoreType.DMA,
        pltpu.SemaphoreType.DMA,
        pltpu.SemaphoreType.DMA,
    ],
)(_dispatch_body)


def _expert_body(te_sref, x_ref, wi_ref, wo_ref, g_ref, y_ref):
    m = pl.program_id(0)

    @pl.when(te_sref[m] < E)
    def _():
        x = x_ref[...].astype(jnp.bfloat16)
        h = jnp.dot(x, wi_ref[0], preferred_element_type=jnp.float32)
        h = jnp.maximum(h, 0.0).astype(jnp.bfloat16)
        y = jnp.dot(h, wo_ref[0], preferred_element_type=jnp.float32)
        y_ref[...] = y * g_ref[...]


def _expert(te, xs, wi, wo, gs):
    return pl.pallas_call(
        _expert_body,
        grid_spec=pltpu.PrefetchScalarGridSpec(
            num_scalar_prefetch=1,
            grid=(NT,),
            in_specs=[
                pl.BlockSpec((TILE_M, D), lambda m, te: (m, 0)),
                pl.BlockSpec((1, D, F), lambda m, te: (te[m] % E, 0, 0)),
                pl.BlockSpec((1, F, D), lambda m, te: (te[m] % E, 0, 0)),
                pl.BlockSpec((TILE_M, 1), lambda m, te: (m, 0)),
            ],
            out_specs=pl.BlockSpec((TILE_M, D), lambda m, te: (m, 0)),
        ),
        out_shape=jax.ShapeDtypeStruct((C, D), jnp.float32),
        compiler_params=pltpu.CompilerParams(vmem_limit_bytes=60000 * 1024),
    )(te, xs, wi, wo, gs)


def _combine_body(y_hbm, inv_hbm, out_hbm,
                  i1_v, i2_v, y1a_v, y2a_v, y1b_v, y2b_v, sem0, sem1):
    wid = _worker_id()
    tbase = wid * TPW
    pltpu.sync_copy(inv_hbm.at[pl.ds(tbase, TPW)], i1_v)
    pltpu.sync_copy(inv_hbm.at[pl.ds(T + tbase, TPW)], i2_v)
    y1 = [y1a_v, y1b_v]
    y2 = [y2a_v, y2b_v]
    sems = [sem0, sem1]
    descs = [None, None]

    def _issue(ch):
        b = ch % 2
        d1 = pltpu.async_copy(
            y_hbm.at[i1_v.at[pl.ds(ch * CH, CH)]], y1[b], sems[b])
        d2 = pltpu.async_copy(
            y_hbm.at[i2_v.at[pl.ds(ch * CH, CH)]], y2[b], sems[b])
        descs[b] = (d1, d2)

    _issue(0)
    for ch in range(NCCH):
        b = ch % 2
        d1, d2 = descs[b]
        d1.wait()
        d2.wait()
        if ch + 1 < NCCH:
            _issue(ch + 1)

        def _row(r, carry):
            for dc in range(D // 16):
                sl = pl.ds(dc * 16, 16)
                y1[b][r, sl] = y1[b][r, sl] + y2[b][r, sl]
            return carry

        lax.fori_loop(0, CH, _row, 0)
        pltpu.sync_copy(y1[b], out_hbm.at[pl.ds(tbase + ch * CH, CH)])


_combine = functools.partial(
    pl.kernel,
    out_type=[jax.ShapeDtypeStruct((T, D), jnp.float32)],
    mesh=_sc_mesh,
    compiler_params=_sc_params,
    scratch_types=[
        pltpu.VMEM((TPW,), jnp.int32),
        pltpu.VMEM((TPW,), jnp.int32),
        pltpu.VMEM((CH, D), jnp.float32),
        pltpu.VMEM((CH, D), jnp.float32),
        pltpu.VMEM((CH, D), jnp.float32),
        pltpu.VMEM((CH, D), jnp.float32),
        pltpu.SemaphoreType.DMA,
        pltpu.SemaphoreType.DMA,
    ],
)(_combine_body)


def kernel(input_batch, W_router, W_in, W_out):
    b, s, d = input_batch.shape
    x = input_batch.reshape(b * s, d)
    e1, e2, w1, w2, pc1, pc2 = _router(x, W_router)
    hist = jnp.concatenate(
        [pc1.reshape(NS, 16), pc2.reshape(NS, 16)], axis=0)
    ea = jnp.concatenate([e1.reshape(T), e2.reshape(T)])
    wa = jnp.concatenate([w1.reshape(T), w2.reshape(T)])
    xs, gs, inv, te = _dispatch(hist, ea, wa, x)
    y = _expert(te[:NT], xs, W_in.astype(jnp.bfloat16),
                W_out.astype(jnp.bfloat16), gs.reshape(C, 1))
    (out,) = _combine(y, inv)
    return out.reshape(b, s, d)


# speedup vs baseline: 1.1267x; 1.1267x over previous
import functools

import jax
import jax.numpy as jnp
from jax import lax
from jax.experimental import pallas as pl
from jax.experimental.pallas import tpu as pltpu
from jax.experimental.pallas import tpu_sc as plsc

E = 8
D = 1024
F = 2048
T = 4096
A = 2 * T
TILE_M = 512
C = A + E * TILE_M
NT = C // TILE_M
NT_PAD = 32
NC, NS = 2, 16
NW = NC * NS
APW = A // NW
TPW = T // NW
RB = 2048
NRB = T // RB
NCHB = RB // 256
XCH = 32
NXCH = APW // XCH
CH = 16
NCCH = TPW // CH

_sc_mesh = plsc.VectorSubcoreMesh(
    core_axis_name="c", subcore_axis_name="s", num_cores=NC, num_subcores=NS)
_sc_params = pltpu.CompilerParams(needs_layout_passes=False)


def _worker_id():
    return lax.axis_index("s") * NC + lax.axis_index("c")


def _router_body(x_ref, wr_ref, e1_ref, e2_ref, w1_ref, w2_ref, h1_ref, h2_ref):
    x = x_ref[...]
    wr = wr_ref[...]
    logits = jnp.dot(x, wr)
    eidx = lax.broadcasted_iota(jnp.int32, (RB, E), 1)
    m1 = jnp.max(logits, axis=1)
    i1 = jnp.min(jnp.where(logits == m1[:, None], eidx, E), axis=1)
    masked = jnp.where(eidx == i1[:, None], -jnp.inf, logits)
    m2 = jnp.max(masked, axis=1)
    i2 = jnp.min(jnp.where(masked == m2[:, None], eidx, E), axis=1)
    t = jnp.exp(m2 - m1)
    g1 = 1.0 / (1.0 + t)
    e1_ref[...] = i1[:, None]
    e2_ref[...] = i2[:, None]
    w1_ref[...] = g1[:, None]
    w2_ref[...] = (t * g1)[:, None]
    lidx = lax.broadcasted_iota(jnp.int32, (RB, 16), 1)
    oh1 = (lidx == i1[:, None]).astype(jnp.int32)
    oh2 = (lidx == i2[:, None]).astype(jnp.int32)
    h1_ref[...] = jnp.concatenate(
        [jnp.sum(oh1[i * 256:(i + 1) * 256], axis=0).reshape(1, 1, 16)
         for i in range(NCHB)], axis=0)
    h2_ref[...] = jnp.concatenate(
        [jnp.sum(oh2[i * 256:(i + 1) * 256], axis=0).reshape(1, 1, 16)
         for i in range(NCHB)], axis=0)


def _router(x, wr):
    return pl.pallas_call(
        _router_body,
        grid=(NRB,),
        in_specs=[
            pl.BlockSpec((RB, D), lambda m: (m, 0)),
            pl.BlockSpec((D, E), lambda m: (0, 0)),
        ],
        out_specs=[
            pl.BlockSpec((RB, 1), lambda m: (m, 0)),
            pl.BlockSpec((RB, 1), lambda m: (m, 0)),
            pl.BlockSpec((RB, 1), lambda m: (m, 0)),
            pl.BlockSpec((RB, 1), lambda m: (m, 0)),
            pl.BlockSpec((NCHB, 1, 16), lambda m: (m, 0, 0)),
            pl.BlockSpec((NCHB, 1, 16), lambda m: (m, 0, 0)),
        ],
        out_shape=[
            jax.ShapeDtypeStruct((T, 1), jnp.int32),
            jax.ShapeDtypeStruct((T, 1), jnp.int32),
            jax.ShapeDtypeStruct((T, 1), jnp.float32),
            jax.ShapeDtypeStruct((T, 1), jnp.float32),
            jax.ShapeDtypeStruct((NCHB * NRB, 1, 16), jnp.int32),
            jax.ShapeDtypeStruct((NCHB * NRB, 1, 16), jnp.int32),
        ],
    )(x, wr)


def _dispatch_body(hist_hbm, ea_hbm, wa_hbm, x_hbm,
                   xs_hbm, gs_hbm, inv_hbm, te_hbm,
                   allhist_v, eid_v, gate_v, cnt_v, te_v,
                   pos8_v, pos_lin_v, xbuf0_v, xbuf1_v, xbuf2_v,
                   rsem0, rsem1, rsem2, ssem0, ssem1, ssem2):
    wid = _worker_id()
    lane = lax.iota(jnp.int32, 16)
    zeros = jnp.zeros((16,), jnp.int32)
    abase = wid * APW
    tok_base = jnp.where(wid < NS, abase, abase - T)

    rdesc = [None, None, None]
    sdesc = [None, None, None]
    bufs = [xbuf0_v, xbuf1_v, xbuf2_v]
    rsems = [rsem0, rsem1, rsem2]
    ssems = [ssem0, ssem1, ssem2]

    def _start_read(c):
        b = c % 3
        if sdesc[b] is not None:
            sdesc[b].wait()
        rdesc[b] = pltpu.async_copy(
            x_hbm.at[pl.ds(tok_base + c * XCH, XCH)], bufs[b], rsems[b])

    _start_read(0)
    _start_read(1)
    _start_read(2)

    pltpu.sync_copy(hist_hbm, allhist_v)
    totals = zeros
    prefix = zeros
    for w in range(NW):
        hv = allhist_v[w]
        totals = totals + hv
        prefix = prefix + jnp.where(w < wid, hv, zeros)
    padded = ((totals + (TILE_M - 1)) // TILE_M) * TILE_M
    seg_end = plsc.cumsum(padded)
    my_base = (seg_end - padded) + prefix

    pltpu.sync_copy(ea_hbm.at[pl.ds(abase, APW)], eid_v)
    pltpu.sync_copy(wa_hbm.at[pl.ds(abase, APW)], gate_v)

    cntvec = my_base
    for g in range(APW // 16):
        cnt_v[...] = cntvec
        v = eid_v[pl.ds(g * 16, 16)]
        base_e = plsc.load_gather(cnt_v, [v])
        ranks = zeros
        for e in range(E):
            m = v == e
            cs = plsc.cumsum(jnp.where(m, 1, 0))
            ranks = jnp.where(m, cs, ranks)
            tote = plsc.all_reduce_population_count(m)
            cntvec = jnp.where(lane == e, cntvec + tote, cntvec)
        pos = base_e + ranks - 1
        pos8_v[g // 2, pl.ds((g % 2) * 16, 16)] = pos
        pos_lin_v[pl.ds(g * 16, 16)] = pos

    pltpu.sync_copy(pos_lin_v, inv_hbm.at[pl.ds(abase, APW)])

    for c in range(NXCH):
        b = c % 3
        rdesc[b].wait()
        sdesc[b] = pltpu.async_copy(bufs[b], xs_hbm.at[pos8_v.at[c]], ssems[b])
        pltpu.sync_copy(
            gate_v.at[pl.ds(c * XCH, XCH)], gs_hbm.at[pos8_v.at[c]])
        if c + 3 < NXCH:
            _start_read(c + 3)
    for b in range(3):
        if sdesc[b] is not None:
            sdesc[b].wait()

    @pl.when(wid == 0)
    def _():
        cnt_v[...] = (seg_end - padded) + totals
        for g in range(NT_PAD // 16):
            jv = (lane + g * 16) * TILE_M
            te = jnp.zeros((16,), jnp.int32)
            for e in range(E):
                se = seg_end[e]
                te = te + jnp.where(jv >= se, 1, 0)
            te = jnp.minimum(te, E - 1)
            dend = plsc.load_gather(cnt_v, [te])
            te_v[pl.ds(g * 16, 16)] = te + jnp.where(jv >= dend, E, 0)
        pltpu.sync_copy(te_v, te_hbm)


_dispatch = functools.partial(
    pl.kernel,
    out_type=[
        jax.ShapeDtypeStruct((C, D), jnp.float32),
        jax.ShapeDtypeStruct((C,), jnp.float32),
        jax.ShapeDtypeStruct((A,), jnp.int32),
        jax.ShapeDtypeStruct((NT_PAD,), jnp.int32),
    ],
    mesh=_sc_mesh,
    compiler_params=_sc_params,
    scratch_types=[
        pltpu.VMEM((NW, 16), jnp.int32),
        pltpu.VMEM((APW,), jnp.int32),
        pltpu.VMEM((APW,), jnp.float32),
        pltpu.VMEM((16,), jnp.int32),
        pltpu.VMEM((NT_PAD,), jnp.int32),
        pltpu.VMEM((NXCH, XCH), jnp.int32),
        pltpu.VMEM((APW,), jnp.int32),
        pltpu.VMEM((XCH, D), jnp.float32),
        pltpu.VMEM((XCH, D), jnp.float32),
        pltpu.VMEM((XCH, D), jnp.float32),
        pltpu.SemaphoreType.DMA,
        pltpu.SemaphoreType.DMA,
        pltpu.SemaphoreType.DMA,
        pltpu.SemaphoreType.DMA,
        pltpu.SemaphoreType.DMA,
        pltpu.SemaphoreType.DMA,
    ],
)(_dispatch_body)


def _expert_body(te_sref, x_ref, wi_ref, wo_ref, g_ref, y_ref):
    m = pl.program_id(0)

    @pl.when(te_sref[m] < E)
    def _():
        x = x_ref[...]
        h = jnp.maximum(jnp.dot(x, wi_ref[0]), 0.0)
        y_ref[...] = jnp.dot(h, wo_ref[0]) * g_ref[...]


def _expert(te, xs, wi, wo, gs):
    return pl.pallas_call(
        _expert_body,
        grid_spec=pltpu.PrefetchScalarGridSpec(
            num_scalar_prefetch=1,
            grid=(NT,),
            in_specs=[
                pl.BlockSpec((TILE_M, D), lambda m, te: (m, 0)),
                pl.BlockSpec((1, D, F), lambda m, te: (te[m] % E, 0, 0)),
                pl.BlockSpec((1, F, D), lambda m, te: (te[m] % E, 0, 0)),
                pl.BlockSpec((TILE_M, 1), lambda m, te: (m, 0)),
            ],
            out_specs=pl.BlockSpec((TILE_M, D), lambda m, te: (m, 0)),
        ),
        out_shape=jax.ShapeDtypeStruct((C, D), jnp.float32),
        compiler_params=pltpu.CompilerParams(vmem_limit_bytes=60000 * 1024),
    )(te, xs, wi, wo, gs)


def _combine_body(y_hbm, inv_hbm, out_hbm,
                  i1_v, i2_v, y1a_v, y2a_v, y1b_v, y2b_v, sem0, sem1):
    wid = _worker_id()
    tbase = wid * TPW
    pltpu.sync_copy(inv_hbm.at[pl.ds(tbase, TPW)], i1_v)
    pltpu.sync_copy(inv_hbm.at[pl.ds(T + tbase, TPW)], i2_v)
    y1 = [y1a_v, y1b_v]
    y2 = [y2a_v, y2b_v]
    sems = [sem0, sem1]
    descs = [None, None]

    def _issue(ch):
        b = ch % 2
        d1 = pltpu.async_copy(
            y_hbm.at[i1_v.at[pl.ds(ch * CH, CH)]], y1[b], sems[b])
        d2 = pltpu.async_copy(
            y_hbm.at[i2_v.at[pl.ds(ch * CH, CH)]], y2[b], sems[b])
        descs[b] = (d1, d2)

    _issue(0)
    for ch in range(NCCH):
        b = ch % 2
        d1, d2 = descs[b]
        d1.wait()
        d2.wait()
        if ch + 1 < NCCH:
            _issue(ch + 1)

        def _row(r, carry):
            for dc in range(D // 16):
                sl = pl.ds(dc * 16, 16)
                y1[b][r, sl] = y1[b][r, sl] + y2[b][r, sl]
            return carry

        lax.fori_loop(0, CH, _row, 0)
        pltpu.sync_copy(y1[b], out_hbm.at[pl.ds(tbase + ch * CH, CH)])


_combine = functools.partial(
    pl.kernel,
    out_type=[jax.ShapeDtypeStruct((T, D), jnp.float32)],
    mesh=_sc_mesh,
    compiler_params=_sc_params,
    scratch_types=[
        pltpu.VMEM((TPW,), jnp.int32),
        pltpu.VMEM((TPW,), jnp.int32),
        pltpu.VMEM((CH, D), jnp.float32),
        pltpu.VMEM((CH, D), jnp.float32),
        pltpu.VMEM((CH, D), jnp.float32),
        pltpu.VMEM((CH, D), jnp.float32),
        pltpu.SemaphoreType.DMA,
        pltpu.SemaphoreType.DMA,
    ],
)(_combine_body)


def kernel(input_batch, W_router, W_in, W_out):
    b, s, d = input_batch.shape
    x = input_batch.reshape(b * s, d)
    e1, e2, w1, w2, pc1, pc2 = _router(x, W_router)
    hist = jnp.concatenate(
        [pc1.reshape(NS, 16), pc2.reshape(NS, 16)], axis=0)
    ea = jnp.concatenate([e1.reshape(T), e2.reshape(T)])
    wa = jnp.concatenate([w1.reshape(T), w2.reshape(T)])
    xs, gs, inv, te = _dispatch(hist, ea, wa, x)
    y = _expert(te[:NT], xs, W_in, W_out, gs.reshape(C, 1))
    (out,) = _combine(y, inv)
    return out.reshape(b, s, d)


# speedup vs baseline: 1.1369x; 1.0090x over previous
import functools

import jax
import jax.numpy as jnp
from jax import lax
from jax.experimental import pallas as pl
from jax.experimental.pallas import tpu as pltpu
from jax.experimental.pallas import tpu_sc as plsc

E = 8
D = 1024
F = 2048
T = 4096
A = 2 * T
TILE_M = 512
C = A + E * TILE_M
NT = C // TILE_M
NT_PAD = 32
NC, NS = 2, 16
NW = NC * NS
APW = A // NW
TPW = T // NW
RB = 2048
NRB = T // RB
NCHB = RB // 256
XCH = 16
NXCH = APW // XCH
NBUF = 6
CH = 16
NCCH = TPW // CH

_sc_mesh = plsc.VectorSubcoreMesh(
    core_axis_name="c", subcore_axis_name="s", num_cores=NC, num_subcores=NS)
_sc_params = pltpu.CompilerParams(needs_layout_passes=False)


def _worker_id():
    return lax.axis_index("s") * NC + lax.axis_index("c")


def _router_body(x_ref, wr_ref, e1_ref, e2_ref, w1_ref, w2_ref, h1_ref, h2_ref):
    x = x_ref[...]
    wr = wr_ref[...]
    logits = jnp.dot(x, wr)
    eidx = lax.broadcasted_iota(jnp.int32, (RB, E), 1)
    m1 = jnp.max(logits, axis=1)
    i1 = jnp.min(jnp.where(logits == m1[:, None], eidx, E), axis=1)
    masked = jnp.where(eidx == i1[:, None], -jnp.inf, logits)
    m2 = jnp.max(masked, axis=1)
    i2 = jnp.min(jnp.where(masked == m2[:, None], eidx, E), axis=1)
    t = jnp.exp(m2 - m1)
    g1 = 1.0 / (1.0 + t)
    e1_ref[...] = i1[:, None]
    e2_ref[...] = i2[:, None]
    w1_ref[...] = g1[:, None]
    w2_ref[...] = (t * g1)[:, None]
    lidx = lax.broadcasted_iota(jnp.int32, (RB, 16), 1)
    oh1 = (lidx == i1[:, None]).astype(jnp.int32)
    oh2 = (lidx == i2[:, None]).astype(jnp.int32)
    h1_ref[...] = jnp.concatenate(
        [jnp.sum(oh1[i * 256:(i + 1) * 256], axis=0).reshape(1, 1, 16)
         for i in range(NCHB)], axis=0)
    h2_ref[...] = jnp.concatenate(
        [jnp.sum(oh2[i * 256:(i + 1) * 256], axis=0).reshape(1, 1, 16)
         for i in range(NCHB)], axis=0)


def _router(x, wr):
    return pl.pallas_call(
        _router_body,
        grid=(NRB,),
        in_specs=[
            pl.BlockSpec((RB, D), lambda m: (m, 0)),
            pl.BlockSpec((D, E), lambda m: (0, 0)),
        ],
        out_specs=[
            pl.BlockSpec((RB, 1), lambda m: (m, 0)),
            pl.BlockSpec((RB, 1), lambda m: (m, 0)),
            pl.BlockSpec((RB, 1), lambda m: (m, 0)),
            pl.BlockSpec((RB, 1), lambda m: (m, 0)),
            pl.BlockSpec((NCHB, 1, 16), lambda m: (m, 0, 0)),
            pl.BlockSpec((NCHB, 1, 16), lambda m: (m, 0, 0)),
        ],
        out_shape=[
            jax.ShapeDtypeStruct((T, 1), jnp.int32),
            jax.ShapeDtypeStruct((T, 1), jnp.int32),
            jax.ShapeDtypeStruct((T, 1), jnp.float32),
            jax.ShapeDtypeStruct((T, 1), jnp.float32),
            jax.ShapeDtypeStruct((NCHB * NRB, 1, 16), jnp.int32),
            jax.ShapeDtypeStruct((NCHB * NRB, 1, 16), jnp.int32),
        ],
    )(x, wr)


def _dispatch_body(hist_hbm, ea_hbm, wa_hbm, x_hbm,
                   xs_hbm, gs_hbm, inv_hbm, te_hbm,
                   allhist_v, eid_v, gate_v, cnt_v, te_v,
                   pos8_v, pos_lin_v, *bufs_and_sems):
    wid = _worker_id()
    lane = lax.iota(jnp.int32, 16)
    zeros = jnp.zeros((16,), jnp.int32)
    abase = wid * APW
    tok_base = jnp.where(wid < NS, abase, abase - T)

    bufs = list(bufs_and_sems[0:NBUF])
    rsems = list(bufs_and_sems[NBUF:2 * NBUF])
    ssems = list(bufs_and_sems[2 * NBUF:3 * NBUF])
    rdesc = [None] * NBUF
    sdesc = [None] * NBUF

    def _start_read(c):
        b = c % NBUF
        if sdesc[b] is not None:
            sdesc[b].wait()
        rdesc[b] = pltpu.async_copy(
            x_hbm.at[pl.ds(tok_base + c * XCH, XCH)], bufs[b], rsems[b])

    for c0 in range(NBUF):
        _start_read(c0)

    pltpu.sync_copy(hist_hbm, allhist_v)
    totals = zeros
    prefix = zeros
    for w in range(NW):
        hv = allhist_v[w]
        totals = totals + hv
        prefix = prefix + jnp.where(w < wid, hv, zeros)
    padded = ((totals + (TILE_M - 1)) // TILE_M) * TILE_M
    seg_end = plsc.cumsum(padded)
    my_base = (seg_end - padded) + prefix

    pltpu.sync_copy(ea_hbm.at[pl.ds(abase, APW)], eid_v)
    pltpu.sync_copy(wa_hbm.at[pl.ds(abase, APW)], gate_v)

    cntvec = my_base
    for g in range(APW // 16):
        cnt_v[...] = cntvec
        v = eid_v[pl.ds(g * 16, 16)]
        base_e = plsc.load_gather(cnt_v, [v])
        ranks = zeros
        for e in range(E):
            m = v == e
            cs = plsc.cumsum(jnp.where(m, 1, 0))
            ranks = jnp.where(m, cs, ranks)
            tote = plsc.all_reduce_population_count(m)
            cntvec = jnp.where(lane == e, cntvec + tote, cntvec)
        pos = base_e + ranks - 1
        pos8_v[(g * 16) // XCH, pl.ds((g * 16) % XCH, 16)] = pos
        pos_lin_v[pl.ds(g * 16, 16)] = pos

    pltpu.sync_copy(pos_lin_v, inv_hbm.at[pl.ds(abase, APW)])

    for c in range(NXCH):
        b = c % NBUF
        rdesc[b].wait()
        sdesc[b] = pltpu.async_copy(bufs[b], xs_hbm.at[pos8_v.at[c]], ssems[b])
        pltpu.sync_copy(
            gate_v.at[pl.ds(c * XCH, XCH)], gs_hbm.at[pos8_v.at[c]])
        if c + NBUF < NXCH:
            _start_read(c + NBUF)
    for b in range(NBUF):
        if sdesc[b] is not None:
            sdesc[b].wait()

    @pl.when(wid == 0)
    def _():
        cnt_v[...] = (seg_end - padded) + totals
        for g in range(NT_PAD // 16):
            jv = (lane + g * 16) * TILE_M
            te = jnp.zeros((16,), jnp.int32)
            for e in range(E):
                se = seg_end[e]
                te = te + jnp.where(jv >= se, 1, 0)
            te = jnp.minimum(te, E - 1)
            dend = plsc.load_gather(cnt_v, [te])
            te_v[pl.ds(g * 16, 16)] = te + jnp.where(jv >= dend, E, 0)
        pltpu.sync_copy(te_v, te_hbm)


_dispatch = functools.partial(
    pl.kernel,
    out_type=[
        jax.ShapeDtypeStruct((C, D), jnp.float32),
        jax.ShapeDtypeStruct((C,), jnp.float32),
        jax.ShapeDtypeStruct((A,), jnp.int32),
        jax.ShapeDtypeStruct((NT_PAD,), jnp.int32),
    ],
    mesh=_sc_mesh,
    compiler_params=_sc_params,
    scratch_types=[
        pltpu.VMEM((NW, 16), jnp.int32),
        pltpu.VMEM((APW,), jnp.int32),
        pltpu.VMEM((APW,), jnp.float32),
        pltpu.VMEM((16,), jnp.int32),
        pltpu.VMEM((NT_PAD,), jnp.int32),
        pltpu.VMEM((NXCH, XCH), jnp.int32),
        pltpu.VMEM((APW,), jnp.int32),
    ] + [pltpu.VMEM((XCH, D), jnp.float32) for _ in range(NBUF)]
      + [pltpu.SemaphoreType.DMA for _ in range(2 * NBUF)],
)(_dispatch_body)


def _expert_body(te_sref, x_ref, wi_ref, wo_ref, g_ref, y_ref):
    m = pl.program_id(0)

    @pl.when(te_sref[m] < E)
    def _():
        x = x_ref[...]
        h = jnp.maximum(jnp.dot(x, wi_ref[0]), 0.0)
        y_ref[...] = jnp.dot(h, wo_ref[0]) * g_ref[...]


def _expert(te, xs, wi, wo, gs):
    return pl.pallas_call(
        _expert_body,
        grid_spec=pltpu.PrefetchScalarGridSpec(
            num_scalar_prefetch=1,
            grid=(NT,),
            in_specs=[
                pl.BlockSpec((TILE_M, D), lambda m, te: (m, 0)),
                pl.BlockSpec((1, D, F), lambda m, te: (te[m] % E, 0, 0)),
                pl.BlockSpec((1, F, D), lambda m, te: (te[m] % E, 0, 0)),
                pl.BlockSpec((TILE_M, 1), lambda m, te: (m, 0)),
            ],
            out_specs=pl.BlockSpec((TILE_M, D), lambda m, te: (m, 0)),
        ),
        out_shape=jax.ShapeDtypeStruct((C, D), jnp.float32),
        compiler_params=pltpu.CompilerParams(vmem_limit_bytes=60000 * 1024),
    )(te, xs, wi, wo, gs)


def _combine_body(y_hbm, inv_hbm, out_hbm,
                  i1_v, i2_v, y1a_v, y2a_v, y1b_v, y2b_v, sem0, sem1):
    wid = _worker_id()
    tbase = wid * TPW
    pltpu.sync_copy(inv_hbm.at[pl.ds(tbase, TPW)], i1_v)
    pltpu.sync_copy(inv_hbm.at[pl.ds(T + tbase, TPW)], i2_v)
    y1 = [y1a_v, y1b_v]
    y2 = [y2a_v, y2b_v]
    sems = [sem0, sem1]
    descs = [None, None]

    def _issue(ch):
        b = ch % 2
        d1 = pltpu.async_copy(
            y_hbm.at[i1_v.at[pl.ds(ch * CH, CH)]], y1[b], sems[b])
        d2 = pltpu.async_copy(
            y_hbm.at[i2_v.at[pl.ds(ch * CH, CH)]], y2[b], sems[b])
        descs[b] = (d1, d2)

    _issue(0)
    for ch in range(NCCH):
        b = ch % 2
        d1, d2 = descs[b]
        d1.wait()
        d2.wait()
        if ch + 1 < NCCH:
            _issue(ch + 1)

        def _row(r, carry):
            for dc in range(D // 16):
                sl = pl.ds(dc * 16, 16)
                y1[b][r, sl] = y1[b][r, sl] + y2[b][r, sl]
            return carry

        lax.fori_loop(0, CH, _row, 0)
        pltpu.sync_copy(y1[b], out_hbm.at[pl.ds(tbase + ch * CH, CH)])


_combine = functools.partial(
    pl.kernel,
    out_type=[jax.ShapeDtypeStruct((T, D), jnp.float32)],
    mesh=_sc_mesh,
    compiler_params=_sc_params,
    scratch_types=[
        pltpu.VMEM((TPW,), jnp.int32),
        pltpu.VMEM((TPW,), jnp.int32),
        pltpu.VMEM((CH, D), jnp.float32),
        pltpu.VMEM((CH, D), jnp.float32),
        pltpu.VMEM((CH, D), jnp.float32),
        pltpu.VMEM((CH, D), jnp.float32),
        pltpu.SemaphoreType.DMA,
        pltpu.SemaphoreType.DMA,
    ],
)(_combine_body)


def kernel(input_batch, W_router, W_in, W_out):
    b, s, d = input_batch.shape
    x = input_batch.reshape(b * s, d)
    e1, e2, w1, w2, pc1, pc2 = _router(x, W_router)
    hist = jnp.concatenate(
        [pc1.reshape(NS, 16), pc2.reshape(NS, 16)], axis=0)
    ea = jnp.concatenate([e1.reshape(T), e2.reshape(T)])
    wa = jnp.concatenate([w1.reshape(T), w2.reshape(T)])
    xs, gs, inv, te = _dispatch(hist, ea, wa, x)
    y = _expert(te[:NT], xs, W_in, W_out, gs.reshape(C, 1))
    (out,) = _combine(y, inv)
    return out.reshape(b, s, d)
